# Initial kernel scaffold; baseline (speedup 1.0000x reference)
import jax
import jax.numpy as jnp
from jax.experimental import pallas as pl


def kernel(pred, target):
    raise NotImplementedError("write your pallas kernel here")


# speedup vs baseline: 14.7425x; 14.7425x over previous
import functools

import jax
import jax.numpy as jnp
from jax.experimental import pallas as pl
from jax.experimental.pallas import tpu as pltpu

_ROWS = 4096
_BLK_C = 128


def _bce(x, z):
    return jnp.maximum(x, 0.0) - x * z + jnp.log1p(jnp.exp(-jnp.abs(x)))


def _order_key(x):
    u = jax.lax.bitcast_convert_type(x, jnp.uint32)
    neg = (u >> 31).astype(jnp.bool_)
    return jnp.where(neg, ~u, u | jnp.uint32(0x80000000))


def _order_key_inv(t):
    u = jnp.where(t >= jnp.uint32(0x80000000), t ^ jnp.uint32(0x80000000), ~t)
    return jax.lax.bitcast_convert_type(u, jnp.float32)


def _select_kernel(pred_ref, tgt_ref, out_ref, key_ref):
    i = pl.program_id(0)
    pred = pred_ref[...]
    tgt = tgt_ref[...]

    pos = jnp.sum(tgt, axis=0, keepdims=True)
    pos_gt = (pos >= (_ROWS / 2)).astype(pred.dtype)
    majority = tgt == pos_gt
    kmin = jnp.sum(jnp.where(majority, 0, 1), axis=0, keepdims=True)

    g = jnp.where(pos_gt > 0.5, -pred, pred)
    ukey = _order_key(g)
    ukey = jnp.where(majority, ukey, jnp.uint32(0))
    key_ref[...] = ukey

    def body(_, carry):
        lo, hi = carry
        mid = lo + ((hi - lo + jnp.uint32(1)) >> 1)
        cnt = jnp.sum(
            jnp.where(key_ref[...] >= mid, 1, 0), axis=0, keepdims=True
        )
        ok = cnt >= kmin
        return jnp.where(ok, mid, lo), jnp.where(ok, hi, mid - jnp.uint32(1))

    lo0 = jnp.zeros((1, _BLK_C), jnp.uint32)
    hi0 = jnp.full((1, _BLK_C), 0xFF800000, jnp.uint32)
    t, _ = jax.lax.fori_loop(0, 32, body, (lo0, hi0))

    ukey = key_ref[...]
    gt = ukey > t
    cnt_gt = jnp.sum(jnp.where(gt, 1, 0), axis=0, keepdims=True)
    loss = _bce(pred, tgt)
    sum_sel = jnp.sum(
        jnp.where(gt | (~majority), loss, 0.0), axis=0, keepdims=True
    )
    gval = _order_key_inv(t)
    pb = jnp.where(pos_gt > 0.5, -gval, gval)
    lossb = _bce(pb, pos_gt)
    tie = (kmin - cnt_gt).astype(jnp.float32)
    csum = jnp.where(kmin > 0, sum_sel + tie * lossb, 0.0)

    @pl.when(i == 0)
    def _():
        out_ref[0, 0] = 0.0

    out_ref[0, 0] += jnp.sum(csum)


@jax.jit
def kernel(pred, target):
    rows, cols = pred.shape
    pad = (-cols) % _BLK_C
    pred_p = jnp.pad(pred, ((0, 0), (0, pad)))
    tgt_p = jnp.pad(target, ((0, 0), (0, pad)))
    cols_p = cols + pad

    total = pl.pallas_call(
        _select_kernel,
        grid=(cols_p // _BLK_C,),
        in_specs=[
            pl.BlockSpec((rows, _BLK_C), lambda i: (0, i)),
            pl.BlockSpec((rows, _BLK_C), lambda i: (0, i)),
        ],
        out_specs=pl.BlockSpec(memory_space=pltpu.SMEM),
        out_shape=jax.ShapeDtypeStruct((1, 1), jnp.float32),
        scratch_shapes=[pltpu.VMEM((rows, _BLK_C), jnp.uint32)],
        compiler_params=pltpu.CompilerParams(
            dimension_semantics=("arbitrary",)
        ),
    )(pred_p, tgt_p)
    return total[0, 0] / (rows * cols)
